# Initial kernel scaffold; baseline (speedup 1.0000x reference)
import jax
import jax.numpy as jnp
from jax.experimental import pallas as pl


def kernel(h, edge_index, W, b):
    raise NotImplementedError("write your pallas kernel here")


# speedup vs baseline: 2.8470x; 2.8470x over previous
import functools

import jax
import jax.numpy as jnp
from jax import lax
from jax.experimental import pallas as pl
from jax.experimental.pallas import tpu as pltpu
from jax.experimental.pallas import tpu_sc as plsc

N = 10000
D = 128
E = 160000

NC = 2
NS = 16
NW = NC * NS
K = 128
CH = 79
SLOTS = NW * CH * K
ROWS = 10240
RPT = ROWS // NS
RCH = RPT // K
TRASH = ROWS - 2

_sc_mesh = plsc.VectorSubcoreMesh(core_axis_name="c", subcore_axis_name="s")


@functools.partial(
    pl.kernel,
    mesh=_sc_mesh,
    out_type=[
        jax.ShapeDtypeStruct((NC * ROWS, D), jnp.float32),
        jax.ShapeDtypeStruct((NC * ROWS,), jnp.float32),
    ],
    scratch_types=[
        pltpu.VMEM((K,), jnp.int32),
        pltpu.VMEM((K,), jnp.int32),
        pltpu.VMEM((K,), jnp.float32),
        pltpu.VMEM((K, D), jnp.float32),
        pltpu.VMEM_SHARED((ROWS, D), jnp.float32),
        pltpu.VMEM_SHARED((ROWS,), jnp.float32),
        pltpu.SemaphoreType.DMA,
    ],
)
def _sc_aggregate(h_hbm, cidx_hbm, ridx_hbm, agg_hbm, deg_hbm,
                  cidx_v, ridx_v, ones_v, rows_v, agg_sh, deg_sh, sem):
    c = lax.axis_index("c")
    s = lax.axis_index("s")
    wid = c * NS + s

    def _zero_row(i, carry):
        for j in range(D // 16):
            rows_v[i, pl.ds(j * 16, 16)] = jnp.zeros((16,), jnp.float32)
        return carry
    lax.fori_loop(0, K, _zero_row, 0)
    for g in range(K // 16):
        ones_v[pl.ds(g * 16, 16)] = jnp.zeros((16,), jnp.float32)

    base_row = s * RPT
    def _zero_chunk(k, carry):
        row0 = base_row + k * K
        for g in range(K // 16):
            ridx_v[pl.ds(g * 16, 16)] = row0 + g * 16 + lax.iota(jnp.int32, 16)
        pltpu.sync_copy(rows_v, agg_sh.at[ridx_v])
        pltpu.sync_copy(ones_v, deg_sh.at[ridx_v])
        return carry
    lax.fori_loop(0, RCH, _zero_chunk, 0)

    for g in range(K // 16):
        ones_v[pl.ds(g * 16, 16)] = jnp.ones((16,), jnp.float32)

    plsc.subcore_barrier()

    base = wid * (CH * K)
    def _chunk(j, carry):
        off = base + j * K
        pltpu.sync_copy(cidx_hbm.at[pl.ds(off, K)], cidx_v)
        pltpu.sync_copy(ridx_hbm.at[pl.ds(off, K)], ridx_v)
        pltpu.async_copy(h_hbm.at[cidx_v], rows_v, sem).wait()
        pltpu.sync_copy(rows_v, agg_sh.at[ridx_v], add=True)
        pltpu.sync_copy(ones_v, deg_sh.at[ridx_v], add=True)
        return carry
    lax.fori_loop(0, CH, _chunk, 0)

    plsc.subcore_barrier()

    def _out_chunk(k, carry):
        row0 = base_row + k * K
        for g in range(K // 16):
            ridx_v[pl.ds(g * 16, 16)] = row0 + g * 16 + lax.iota(jnp.int32, 16)
        pltpu.sync_copy(agg_sh.at[ridx_v], rows_v)
        pltpu.sync_copy(rows_v, agg_hbm.at[pl.ds(c * ROWS + row0, K)])
        pltpu.sync_copy(deg_sh.at[ridx_v], ones_v)
        pltpu.sync_copy(ones_v, deg_hbm.at[pl.ds(c * ROWS + row0, K)])
        return carry
    lax.fori_loop(0, RCH, _out_chunk, 0)


_BR = 400


def _tc_body(h_ref, a0_ref, a1_ref, d0_ref, d1_ref, w_ref, b_ref, o_ref):
    a = a0_ref[...] + a1_ref[...]
    deg = d0_ref[...] + d1_ref[...]
    agg = jnp.where(deg > 0.0, a / jnp.maximum(deg, 1.0), 0.0)
    x = h_ref[...] + agg
    o_ref[...] = lax.dot_general(
        x, w_ref[...], (((1,), (1,)), ((), ())),
        precision=lax.Precision.HIGHEST) + b_ref[...]


def _tc_linear(h, a0, a1, d0, d1, W, b2):
    grid = (N // _BR,)
    return pl.pallas_call(
        _tc_body,
        grid=grid,
        in_specs=[
            pl.BlockSpec((_BR, D), lambda i: (i, 0)),
            pl.BlockSpec((_BR, D), lambda i: (i, 0)),
            pl.BlockSpec((_BR, D), lambda i: (i, 0)),
            pl.BlockSpec((_BR, 1), lambda i: (i, 0)),
            pl.BlockSpec((_BR, 1), lambda i: (i, 0)),
            pl.BlockSpec((D, D), lambda i: (0, 0)),
            pl.BlockSpec((1, D), lambda i: (0, 0)),
        ],
        out_specs=pl.BlockSpec((_BR, D), lambda i: (i, 0)),
        out_shape=jax.ShapeDtypeStruct((N, D), jnp.float32),
    )(h, a0, a1, d0, d1, W, b2)


def kernel(h, edge_index, W, b):
    e = edge_index.astype(jnp.int32)
    keys = jnp.sort(jnp.concatenate([e[0] * N + e[1], e[1] * N + e[0]]))
    valid = jnp.concatenate(
        [jnp.ones((1,), jnp.bool_), keys[1:] != keys[:-1]])
    r = keys // N
    cc = keys - r * N
    ridx = jnp.where(valid, r, TRASH)
    cidx = jnp.where(valid, cc, 0)
    pad = SLOTS - 2 * E
    ridx = jnp.concatenate([ridx, jnp.full((pad,), TRASH, jnp.int32)])
    cidx = jnp.concatenate([cidx, jnp.zeros((pad,), jnp.int32)])

    agg_flat, deg_flat = _sc_aggregate(h, cidx, ridx)

    agg2 = agg_flat.reshape(NC, ROWS, D)
    deg2 = deg_flat.reshape(NC, ROWS)[:, :N]
    return _tc_linear(h, agg2[0, :N], agg2[1, :N],
                      deg2[0].reshape(N, 1), deg2[1].reshape(N, 1),
                      W, b.reshape(1, D))
